# baseline (speedup 1.0000x reference)
import functools
import jax
import jax.numpy as jnp
from jax import lax
from jax.experimental import pallas as pl
from jax.experimental.pallas import tpu as pltpu
from jax.experimental.pallas import tpu_sc as plsc

_LANES = 16


def _bias_body(x_ref, w1_ref, b1_ref, w2_ref, b2_ref, out_ref):
    h = jnp.maximum(
        jnp.dot(x_ref[...], w1_ref[...], preferred_element_type=jnp.float32)
        + b1_ref[...], 0.0)
    out_ref[...] = (
        jnp.dot(h, w2_ref[...], preferred_element_type=jnp.float32) + b2_ref[...])


def kernel(label, image_features, ctx, W1, b1, W2, b2, token_prefix, token_suffix):
    B = label.shape[0]
    num_classes, n_ctx, ctx_dim = ctx.shape
    vis_dim = image_features.shape[1]
    hid = W1.shape[1]
    pre_len = token_prefix.shape[1]
    suf_len = token_suffix.shape[1]
    seq = pre_len + n_ctx + suf_len
    row = seq * ctx_dim
    pre_n = pre_len * ctx_dim
    mid_n = n_ctx * ctx_dim
    suf_n = suf_len * ctx_dim

    bias = pl.pallas_call(
        _bias_body,
        out_shape=jax.ShapeDtypeStruct((B, ctx_dim), jnp.float32),
    )(image_features, W1, b1.reshape(1, hid), W2, b2.reshape(1, ctx_dim))

    ctx2d = ctx.reshape(num_classes, mid_n)
    pre1 = token_prefix.reshape(pre_n)
    suf1 = token_suffix.reshape(suf_n)

    info = plsc.get_sparse_core_info()
    nw = info.num_cores * info.num_subcores
    b_per_w = B // nw
    mesh = plsc.VectorSubcoreMesh(core_axis_name="c", subcore_axis_name="s")

    @functools.partial(
        pl.kernel,
        out_type=jax.ShapeDtypeStruct((B, row), jnp.float32),
        mesh=mesh,
        scratch_types=[
            pltpu.VMEM((b_per_w,), jnp.int32),
            pltpu.VMEM((b_per_w, ctx_dim), jnp.float32),
            pltpu.VMEM((b_per_w, mid_n), jnp.float32),
            pltpu.VMEM((row,), jnp.float32),
            pltpu.SemaphoreType.DMA,
        ],
    )
    def sc_gather(ctx_hbm, lbl_hbm, bias_hbm, pre_hbm, suf_hbm, out_hbm,
                  idx_v, bias_v, rows_v, row_v, sem):
        wid = lax.axis_index("s") * info.num_cores + lax.axis_index("c")
        base = wid * b_per_w
        pltpu.sync_copy(lbl_hbm.at[pl.ds(base, b_per_w)], idx_v)
        pltpu.sync_copy(bias_hbm.at[pl.ds(base, b_per_w)], bias_v)
        pltpu.sync_copy(pre_hbm, row_v.at[pl.ds(0, pre_n)])
        pltpu.sync_copy(suf_hbm, row_v.at[pl.ds(pre_n + mid_n, suf_n)])
        pltpu.async_copy(ctx_hbm.at[idx_v], rows_v, sem).wait()

        def body(b, carry):
            for k in range(mid_n // _LANES):
                koff = (k * _LANES) % ctx_dim
                row_v[pl.ds(pre_n + k * _LANES, _LANES)] = (
                    rows_v[b, pl.ds(k * _LANES, _LANES)]
                    + bias_v[b, pl.ds(koff, _LANES)])
            pltpu.sync_copy(row_v, out_hbm.at[base + b])
            return carry

        lax.fori_loop(0, b_per_w, body, 0)

    out2d = sc_gather(ctx2d, label.astype(jnp.int32), bias, pre1, suf1)
    return out2d.reshape(B, seq, ctx_dim)


# speedup vs baseline: 3.2675x; 3.2675x over previous
import functools
import jax
import jax.numpy as jnp
from jax import lax
from jax.experimental import pallas as pl
from jax.experimental.pallas import tpu as pltpu
from jax.experimental.pallas import tpu_sc as plsc

_BB = 64


def _mid_body(lbl_ref, x_ref, w1_ref, b1_ref, w2_ref, b2_ref,
              ctx_any, g_ref, gbuf, gsem):
    nb = pl.num_programs(0)
    i = pl.program_id(0)
    slot = jax.lax.rem(i, 2)

    def start(s, step):
        for j in range(_BB):
            pltpu.make_async_copy(
                ctx_any.at[lbl_ref[step * _BB + j]],
                gbuf.at[s, j],
                gsem.at[s, j],
            ).start()

    @pl.when(i == 0)
    def _():
        start(0, 0)

    @pl.when(i + 1 < nb)
    def _():
        start(1 - slot, i + 1)

    for j in range(_BB):
        pltpu.make_async_copy(ctx_any.at[0], gbuf.at[slot, j],
                              gsem.at[slot, j]).wait()

    h = jnp.maximum(
        jnp.dot(x_ref[...], w1_ref[...], preferred_element_type=jnp.float32)
        + b1_ref[...], 0.0)
    bias = jnp.dot(h, w2_ref[...], preferred_element_type=jnp.float32) + b2_ref[...]

    ctx_sel = gbuf[slot]
    for r in range(gbuf.shape[2]):
        g_ref[r] = ctx_sel[:, r, :] + bias


def kernel(label, image_features, ctx, W1, b1, W2, b2, token_prefix, token_suffix):
    B = label.shape[0]
    num_classes, n_ctx, ctx_dim = ctx.shape
    vis_dim = image_features.shape[1]
    hid = W1.shape[1]
    pre_len = token_prefix.shape[1]
    suf_len = token_suffix.shape[1]
    seq = pre_len + n_ctx + suf_len
    n_tok = pre_len + suf_len
    nb = B // _BB

    grid_spec = pltpu.PrefetchScalarGridSpec(
        num_scalar_prefetch=1,
        grid=(nb,),
        in_specs=[
            pl.BlockSpec((_BB, vis_dim), lambda i, lbl: (i, 0)),
            pl.BlockSpec((vis_dim, hid), lambda i, lbl: (0, 0)),
            pl.BlockSpec((1, hid), lambda i, lbl: (0, 0)),
            pl.BlockSpec((hid, ctx_dim), lambda i, lbl: (0, 0)),
            pl.BlockSpec((1, ctx_dim), lambda i, lbl: (0, 0)),
            pl.BlockSpec(memory_space=pl.ANY),
        ],
        out_specs=pl.BlockSpec((n_ctx, _BB, ctx_dim), lambda i, lbl: (0, i, 0)),
        scratch_shapes=[
            pltpu.VMEM((2, _BB, n_ctx, ctx_dim), jnp.float32),
            pltpu.SemaphoreType.DMA((2, _BB)),
        ],
    )

    g = pl.pallas_call(
        _mid_body,
        grid_spec=grid_spec,
        out_shape=jax.ShapeDtypeStruct((n_ctx, B, ctx_dim), jnp.float32),
    )(label.astype(jnp.int32), image_features, W1, b1.reshape(1, hid), W2,
      b2.reshape(1, ctx_dim), ctx)

    tokens = jnp.concatenate(
        [token_prefix.reshape(pre_len, ctx_dim),
         token_suffix.reshape(suf_len, ctx_dim)], axis=0)

    info = plsc.get_sparse_core_info()
    nw = info.num_cores * info.num_subcores
    n_units = 2 * (n_tok + n_ctx)
    half = B // 2
    blk = 32
    k_per_half = half // blk
    mesh = plsc.VectorSubcoreMesh(core_axis_name="c", subcore_axis_name="s")

    @functools.partial(
        pl.kernel,
        out_type=jax.ShapeDtypeStruct((seq, B, ctx_dim), jnp.float32),
        mesh=mesh,
        scratch_types=[
            pltpu.VMEM((blk, ctx_dim), jnp.float32),
            pltpu.VMEM((blk, ctx_dim), jnp.float32),
            pltpu.SemaphoreType.DMA,
            pltpu.SemaphoreType.DMA,
        ],
    )
    def sc_fill(g_hbm, tok_hbm, out_hbm, pat_v, mid_v, wsem, msem):
        wid = lax.axis_index("s") * info.num_cores + lax.axis_index("c")
        u_lo = wid * n_units // nw
        u_hi = (wid + 1) * n_units // nw

        def unit(u, carry):
            is_mid = u < 2 * n_ctx

            @pl.when(is_mid)
            def _():
                r = u // 2
                h = u - 2 * r
                for k in range(k_per_half):
                    off = h * half + k * blk
                    pltpu.sync_copy(g_hbm.at[r, pl.ds(off, blk)], mid_v)
                    pltpu.async_copy(mid_v, out_hbm.at[pre_len + r, pl.ds(off, blk)],
                                     msem).wait()

            @pl.when(jnp.logical_not(is_mid))
            def _():
                ut = u - 2 * n_ctx
                st = ut // 2
                h = ut - 2 * st
                s_out = jnp.where(st < pre_len, st, st + n_ctx)
                for row in range(blk):
                    pltpu.async_copy(tok_hbm.at[st], pat_v.at[row], msem).start()
                for row in range(blk):
                    pltpu.make_async_copy(tok_hbm.at[st], pat_v.at[row],
                                          msem).wait()
                for k in range(k_per_half):
                    off = h * half + k * blk
                    pltpu.async_copy(pat_v, out_hbm.at[s_out, pl.ds(off, blk)],
                                     wsem).start()
                for k in range(k_per_half):
                    pltpu.make_async_copy(pat_v, out_hbm.at[s_out, pl.ds(0, blk)],
                                          wsem).wait()

            return carry

        lax.fori_loop(u_lo, u_hi, unit, 0)

    out3 = sc_fill(g, tokens)
    return out3.transpose(1, 0, 2)


# speedup vs baseline: 5.0412x; 1.5428x over previous
import functools
import jax
import jax.numpy as jnp
from jax import lax
from jax.experimental import pallas as pl
from jax.experimental.pallas import tpu as pltpu
from jax.experimental.pallas import tpu_sc as plsc

_BB = 64


def _mid_body(lbl_ref, x_ref, w1_ref, b1_ref, w2_ref, b2_ref,
              ctx_any, g_ref, gbuf, gsem):
    nb = pl.num_programs(0)
    i = pl.program_id(0)
    slot = jax.lax.rem(i, 2)

    def start(s, step):
        for j in range(_BB):
            pltpu.make_async_copy(
                ctx_any.at[lbl_ref[step * _BB + j]],
                gbuf.at[s, j],
                gsem.at[s, j],
            ).start()

    @pl.when(i == 0)
    def _():
        start(0, 0)

    @pl.when(i + 1 < nb)
    def _():
        start(1 - slot, i + 1)

    for j in range(_BB):
        pltpu.make_async_copy(ctx_any.at[0], gbuf.at[slot, j],
                              gsem.at[slot, j]).wait()

    h = jnp.maximum(
        jnp.dot(x_ref[...], w1_ref[...], preferred_element_type=jnp.float32)
        + b1_ref[...], 0.0)
    bias = jnp.dot(h, w2_ref[...], preferred_element_type=jnp.float32) + b2_ref[...]

    ctx_sel = gbuf[slot]
    for r in range(gbuf.shape[2]):
        g_ref[r] = ctx_sel[:, r, :] + bias


def _rep_body(tok_ref, out_ref):
    out_ref[0] = jnp.broadcast_to(tok_ref[0], out_ref.shape[1:])


def kernel(label, image_features, ctx, W1, b1, W2, b2, token_prefix, token_suffix):
    B = label.shape[0]
    num_classes, n_ctx, ctx_dim = ctx.shape
    vis_dim = image_features.shape[1]
    hid = W1.shape[1]
    pre_len = token_prefix.shape[1]
    suf_len = token_suffix.shape[1]
    seq = pre_len + n_ctx + suf_len
    n_tok = pre_len + suf_len
    nb = B // _BB

    grid_spec = pltpu.PrefetchScalarGridSpec(
        num_scalar_prefetch=1,
        grid=(nb,),
        in_specs=[
            pl.BlockSpec((_BB, vis_dim), lambda i, lbl: (i, 0)),
            pl.BlockSpec((vis_dim, hid), lambda i, lbl: (0, 0)),
            pl.BlockSpec((1, hid), lambda i, lbl: (0, 0)),
            pl.BlockSpec((hid, ctx_dim), lambda i, lbl: (0, 0)),
            pl.BlockSpec((1, ctx_dim), lambda i, lbl: (0, 0)),
            pl.BlockSpec(memory_space=pl.ANY),
        ],
        out_specs=pl.BlockSpec((n_ctx, _BB, ctx_dim), lambda i, lbl: (0, i, 0)),
        scratch_shapes=[
            pltpu.VMEM((2, _BB, n_ctx, ctx_dim), jnp.float32),
            pltpu.SemaphoreType.DMA((2, _BB)),
        ],
    )

    g = pl.pallas_call(
        _mid_body,
        grid_spec=grid_spec,
        out_shape=jax.ShapeDtypeStruct((n_ctx, B, ctx_dim), jnp.float32),
    )(label.astype(jnp.int32), image_features, W1, b1.reshape(1, hid), W2,
      b2.reshape(1, ctx_dim), ctx)

    tokens = jnp.concatenate(
        [token_prefix.reshape(pre_len, ctx_dim),
         token_suffix.reshape(suf_len, ctx_dim)], axis=0)

    rep = 128
    rep_grid = pltpu.PrefetchScalarGridSpec(
        num_scalar_prefetch=0,
        grid=(n_tok,),
        in_specs=[pl.BlockSpec((1, 1, ctx_dim), lambda i: (i, 0, 0))],
        out_specs=pl.BlockSpec((1, rep, ctx_dim), lambda i: (i, 0, 0)),
    )
    tokens_rep = pl.pallas_call(
        _rep_body,
        grid_spec=rep_grid,
        out_shape=jax.ShapeDtypeStruct((n_tok, rep, ctx_dim), jnp.float32),
    )(tokens.reshape(n_tok, 1, ctx_dim))

    info = plsc.get_sparse_core_info()
    nw = info.num_cores * info.num_subcores
    n_units = 2 * (n_tok + n_ctx)
    half = B // 2
    blk = rep
    k_per_half = half // blk
    mesh = plsc.VectorSubcoreMesh(core_axis_name="c", subcore_axis_name="s")

    @functools.partial(
        pl.kernel,
        out_type=jax.ShapeDtypeStruct((seq, B, ctx_dim), jnp.float32),
        mesh=mesh,
        scratch_types=[
            pltpu.VMEM((blk, ctx_dim), jnp.float32),
            pltpu.SemaphoreType.DMA,
            pltpu.SemaphoreType.DMA,
        ],
    )
    def sc_fill(g_hbm, rep_hbm, out_hbm, pat_v, wsem, msem):
        wid = lax.axis_index("s") * info.num_cores + lax.axis_index("c")
        u_lo = wid * n_units // nw
        u_hi = (wid + 1) * n_units // nw

        def unit(u, carry):
            is_mid = u < 2 * n_ctx
            r = u // 2
            hm = u - 2 * r
            ut = jnp.maximum(u - 2 * n_ctx, 0)
            st = ut // 2
            h = ut - 2 * st
            s_out = jnp.where(st < pre_len, st, st + n_ctx)

            @pl.when(is_mid)
            def _():
                for k in range(k_per_half):
                    off = hm * half + k * blk
                    pltpu.sync_copy(g_hbm.at[r, pl.ds(off, blk)], pat_v)
                    pltpu.async_copy(pat_v,
                                     out_hbm.at[pre_len + r, pl.ds(off, blk)],
                                     msem).wait()

            @pl.when(jnp.logical_not(is_mid))
            def _():
                pltpu.sync_copy(rep_hbm.at[st], pat_v)
                for k in range(k_per_half):
                    off = h * half + k * blk
                    pltpu.async_copy(pat_v, out_hbm.at[s_out, pl.ds(off, blk)],
                                     wsem).wait()

            return carry

        lax.fori_loop(u_lo, u_hi, unit, 0)

    out3 = sc_fill(g, tokens_rep)
    return out3.transpose(1, 0, 2)


# speedup vs baseline: 6.6043x; 1.3101x over previous
import functools
import jax
import jax.numpy as jnp
from jax import lax
from jax.experimental import pallas as pl
from jax.experimental.pallas import tpu as pltpu
from jax.experimental.pallas import tpu_sc as plsc

_BB = 64


def _mid_body(lbl_ref, x_ref, w1_ref, b1_ref, w2_ref, b2_ref,
              ctx_any, g_ref, gbuf, gsem):
    nb = pl.num_programs(0)
    i = pl.program_id(0)
    slot = jax.lax.rem(i, 2)

    def start(s, step):
        for j in range(_BB):
            pltpu.make_async_copy(
                ctx_any.at[lbl_ref[step * _BB + j]],
                gbuf.at[s, j],
                gsem.at[s, j],
            ).start()

    @pl.when(i == 0)
    def _():
        start(0, 0)

    @pl.when(i + 1 < nb)
    def _():
        start(1 - slot, i + 1)

    for j in range(_BB):
        pltpu.make_async_copy(ctx_any.at[0], gbuf.at[slot, j],
                              gsem.at[slot, j]).wait()

    h = jnp.maximum(
        jnp.dot(x_ref[...], w1_ref[...], preferred_element_type=jnp.float32)
        + b1_ref[...], 0.0)
    bias = jnp.dot(h, w2_ref[...], preferred_element_type=jnp.float32) + b2_ref[...]

    ctx_sel = gbuf[slot]
    for r in range(gbuf.shape[2]):
        g_ref[r] = ctx_sel[:, r, :] + bias


def _rep_body(tok_ref, out_ref):
    out_ref[0] = jnp.broadcast_to(tok_ref[0], out_ref.shape[1:])


def kernel(label, image_features, ctx, W1, b1, W2, b2, token_prefix, token_suffix):
    B = label.shape[0]
    num_classes, n_ctx, ctx_dim = ctx.shape
    vis_dim = image_features.shape[1]
    hid = W1.shape[1]
    pre_len = token_prefix.shape[1]
    suf_len = token_suffix.shape[1]
    seq = pre_len + n_ctx + suf_len
    n_tok = pre_len + suf_len
    nb = B // _BB

    grid_spec = pltpu.PrefetchScalarGridSpec(
        num_scalar_prefetch=1,
        grid=(nb,),
        in_specs=[
            pl.BlockSpec((_BB, vis_dim), lambda i, lbl: (i, 0)),
            pl.BlockSpec((vis_dim, hid), lambda i, lbl: (0, 0)),
            pl.BlockSpec((1, hid), lambda i, lbl: (0, 0)),
            pl.BlockSpec((hid, ctx_dim), lambda i, lbl: (0, 0)),
            pl.BlockSpec((1, ctx_dim), lambda i, lbl: (0, 0)),
            pl.BlockSpec(memory_space=pl.ANY),
        ],
        out_specs=pl.BlockSpec((n_ctx, _BB, ctx_dim), lambda i, lbl: (0, i, 0)),
        scratch_shapes=[
            pltpu.VMEM((2, _BB, n_ctx, ctx_dim), jnp.float32),
            pltpu.SemaphoreType.DMA((2, _BB)),
        ],
    )

    g = pl.pallas_call(
        _mid_body,
        grid_spec=grid_spec,
        out_shape=jax.ShapeDtypeStruct((n_ctx, B, ctx_dim), jnp.float32),
    )(label.astype(jnp.int32), image_features, W1, b1.reshape(1, hid), W2,
      b2.reshape(1, ctx_dim), ctx)

    tokens = jnp.concatenate(
        [token_prefix.reshape(pre_len, ctx_dim),
         token_suffix.reshape(suf_len, ctx_dim)], axis=0)

    rep = 128
    rep_grid = pltpu.PrefetchScalarGridSpec(
        num_scalar_prefetch=0,
        grid=(n_tok,),
        in_specs=[pl.BlockSpec((1, 1, ctx_dim), lambda i: (i, 0, 0))],
        out_specs=pl.BlockSpec((1, rep, ctx_dim), lambda i: (i, 0, 0)),
    )
    tokens_rep = pl.pallas_call(
        _rep_body,
        grid_spec=rep_grid,
        out_shape=jax.ShapeDtypeStruct((n_tok, rep, ctx_dim), jnp.float32),
    )(tokens.reshape(n_tok, 1, ctx_dim))

    info = plsc.get_sparse_core_info()
    nw = info.num_cores * info.num_subcores
    n_units = 2 * n_tok
    half = B // 2
    blk = rep
    k_per_half = half // blk
    mesh = plsc.VectorSubcoreMesh(core_axis_name="c", subcore_axis_name="s")

    @functools.partial(
        pl.kernel,
        out_type=jax.ShapeDtypeStruct((seq, B, ctx_dim), jnp.float32),
        mesh=mesh,
        scratch_types=[
            pltpu.VMEM((blk, ctx_dim), jnp.float32),
            pltpu.SemaphoreType.DMA,
        ],
    )
    def sc_fill(rep_hbm, out_hbm, pat_v, wsem):
        wid = lax.axis_index("s") * info.num_cores + lax.axis_index("c")
        u_lo = wid * n_units // nw
        u_hi = (wid + 1) * n_units // nw

        def unit(u, carry):
            st = u // 2
            h = u - 2 * st
            s_out = jnp.where(st < pre_len, st, st + n_ctx)
            pltpu.sync_copy(rep_hbm.at[st], pat_v)
            for k in range(k_per_half):
                off = h * half + k * blk
                pltpu.async_copy(pat_v, out_hbm.at[s_out, pl.ds(off, blk)],
                                 wsem).wait()
            return carry

        lax.fori_loop(u_lo, u_hi, unit, 0)

    out3 = sc_fill(tokens_rep)

    bb2 = 128
    nb2 = B // bb2

    def _merge_body(g_ref, in_any, out_any, sem):
        i = pl.program_id(0)
        pltpu.async_copy(
            g_ref,
            out_any.at[pl.ds(pre_len, n_ctx), pl.ds(i * bb2, bb2)],
            sem).wait()

    merged = pl.pallas_call(
        _merge_body,
        grid=(nb2,),
        in_specs=[
            pl.BlockSpec((n_ctx, bb2, ctx_dim), lambda i: (0, i, 0)),
            pl.BlockSpec(memory_space=pl.ANY),
        ],
        out_specs=pl.BlockSpec(memory_space=pl.ANY),
        out_shape=jax.ShapeDtypeStruct((seq, B, ctx_dim), jnp.float32),
        input_output_aliases={1: 0},
        scratch_shapes=[pltpu.SemaphoreType.DMA],
    )(g, out3)
    return merged.transpose(1, 0, 2)


# speedup vs baseline: 6.9129x; 1.0467x over previous
import functools
import jax
import jax.numpy as jnp
from jax import lax
from jax.experimental import pallas as pl
from jax.experimental.pallas import tpu as pltpu
from jax.experimental.pallas import tpu_sc as plsc

_BB = 64


def _mid_body(lbl_ref, x_ref, w1_ref, b1_ref, w2_ref, b2_ref,
              ctx_any, g_ref, gbuf, gsem):
    nb = pl.num_programs(0)
    i = pl.program_id(0)
    slot = jax.lax.rem(i, 2)

    def start(s, step):
        for j in range(_BB):
            pltpu.make_async_copy(
                ctx_any.at[lbl_ref[step * _BB + j]],
                gbuf.at[s, j],
                gsem.at[s, j],
            ).start()

    @pl.when(i == 0)
    def _():
        start(0, 0)

    @pl.when(i + 1 < nb)
    def _():
        start(1 - slot, i + 1)

    for j in range(_BB):
        pltpu.make_async_copy(ctx_any.at[0], gbuf.at[slot, j],
                              gsem.at[slot, j]).wait()

    h = jnp.maximum(
        jnp.dot(x_ref[...], w1_ref[...], preferred_element_type=jnp.float32)
        + b1_ref[...], 0.0)
    bias = jnp.dot(h, w2_ref[...], preferred_element_type=jnp.float32) + b2_ref[...]

    ctx_sel = gbuf[slot]
    for r in range(gbuf.shape[2]):
        g_ref[r] = ctx_sel[:, r, :] + bias


def _rep_body(tok_ref, out_ref):
    out_ref[0] = jnp.broadcast_to(tok_ref[0], out_ref.shape[1:])


def kernel(label, image_features, ctx, W1, b1, W2, b2, token_prefix, token_suffix):
    B = label.shape[0]
    num_classes, n_ctx, ctx_dim = ctx.shape
    vis_dim = image_features.shape[1]
    hid = W1.shape[1]
    pre_len = token_prefix.shape[1]
    suf_len = token_suffix.shape[1]
    seq = pre_len + n_ctx + suf_len
    n_tok = pre_len + suf_len
    nb = B // _BB

    grid_spec = pltpu.PrefetchScalarGridSpec(
        num_scalar_prefetch=1,
        grid=(nb,),
        in_specs=[
            pl.BlockSpec((_BB, vis_dim), lambda i, lbl: (i, 0)),
            pl.BlockSpec((vis_dim, hid), lambda i, lbl: (0, 0)),
            pl.BlockSpec((1, hid), lambda i, lbl: (0, 0)),
            pl.BlockSpec((hid, ctx_dim), lambda i, lbl: (0, 0)),
            pl.BlockSpec((1, ctx_dim), lambda i, lbl: (0, 0)),
            pl.BlockSpec(memory_space=pl.ANY),
        ],
        out_specs=pl.BlockSpec((n_ctx, _BB, ctx_dim), lambda i, lbl: (0, i, 0)),
        scratch_shapes=[
            pltpu.VMEM((2, _BB, n_ctx, ctx_dim), jnp.float32),
            pltpu.SemaphoreType.DMA((2, _BB)),
        ],
    )

    g = pl.pallas_call(
        _mid_body,
        grid_spec=grid_spec,
        out_shape=jax.ShapeDtypeStruct((n_ctx, B, ctx_dim), jnp.float32),
    )(label.astype(jnp.int32), image_features, W1, b1.reshape(1, hid), W2,
      b2.reshape(1, ctx_dim), ctx)

    tokens = jnp.concatenate(
        [token_prefix.reshape(pre_len, ctx_dim),
         token_suffix.reshape(suf_len, ctx_dim)], axis=0)

    rep = 128
    rep_grid = pltpu.PrefetchScalarGridSpec(
        num_scalar_prefetch=0,
        grid=(n_tok,),
        in_specs=[pl.BlockSpec((1, 1, ctx_dim), lambda i: (i, 0, 0))],
        out_specs=pl.BlockSpec((1, rep, ctx_dim), lambda i: (i, 0, 0)),
    )
    tokens_rep = pl.pallas_call(
        _rep_body,
        grid_spec=rep_grid,
        out_shape=jax.ShapeDtypeStruct((n_tok, rep, ctx_dim), jnp.float32),
    )(tokens.reshape(n_tok, 1, ctx_dim))

    info = plsc.get_sparse_core_info()
    nw = info.num_cores * info.num_subcores
    n_units = 2 * n_tok
    half = B // 2
    blk = rep
    k_per_half = half // blk
    mesh = plsc.VectorSubcoreMesh(core_axis_name="c", subcore_axis_name="s")

    @functools.partial(
        pl.kernel,
        out_type=jax.ShapeDtypeStruct((seq, B, ctx_dim), jnp.float32),
        mesh=mesh,
        scratch_types=[
            pltpu.VMEM((blk, ctx_dim), jnp.float32),
            pltpu.SemaphoreType.DMA,
        ],
    )
    def sc_fill(rep_hbm, out_hbm, pat_v, wsem):
        wid = lax.axis_index("s") * info.num_cores + lax.axis_index("c")
        u_lo = wid * n_units // nw
        u_hi = (wid + 1) * n_units // nw

        def unit(u, carry):
            st = u // 2
            h = u - 2 * st
            s_out = jnp.where(st < pre_len, st, st + n_ctx)

            @pl.when(jnp.logical_or(u == u_lo, h == 0))
            def _():
                pltpu.sync_copy(rep_hbm.at[st], pat_v)
            for k in range(k_per_half):
                off = h * half + k * blk
                pltpu.async_copy(pat_v, out_hbm.at[s_out, pl.ds(off, blk)],
                                 wsem).wait()
            return carry

        lax.fori_loop(u_lo, u_hi, unit, 0)

    out3 = sc_fill(tokens_rep)

    bb2 = 128
    nb2 = B // bb2

    def _merge_body(g_ref, in_any, out_any, sem):
        i = pl.program_id(0)
        pltpu.async_copy(
            g_ref,
            out_any.at[pl.ds(pre_len, n_ctx), pl.ds(i * bb2, bb2)],
            sem).wait()

    merged = pl.pallas_call(
        _merge_body,
        grid=(nb2,),
        in_specs=[
            pl.BlockSpec((n_ctx, bb2, ctx_dim), lambda i: (0, i, 0)),
            pl.BlockSpec(memory_space=pl.ANY),
        ],
        out_specs=pl.BlockSpec(memory_space=pl.ANY),
        out_shape=jax.ShapeDtypeStruct((seq, B, ctx_dim), jnp.float32),
        input_output_aliases={1: 0},
        scratch_shapes=[pltpu.SemaphoreType.DMA],
    )(g, out3)
    return merged.transpose(1, 0, 2)
